# baseline (speedup 1.0000x reference)
import functools

import jax
import jax.numpy as jnp
from jax import lax
from jax.experimental import pallas as pl
from jax.experimental.pallas import tpu as pltpu
from jax.experimental.pallas import tpu_sc as plsc

VOCAB = 100000
EMBED_DIM = 300
BATCH = 4096
HIST = 50

NUM_CORES = 2
NUM_SUBCORES = 16
NW = NUM_CORES * NUM_SUBCORES

VPAD = 100096
DPAD = 304
NDT = DPAD // 8
NVT = VPAD // 128
LASTW = VOCAB - 128 * (NVT - 1)
HALF_ROWS = 64 * DPAD // 128
LAST_WB_ROWS = 80
NBT = BATCH // 128
NHT = 7

STRIPS_PER_W = 25

_mesh = plsc.VectorSubcoreMesh(core_axis_name="c", subcore_axis_name="s")


@functools.partial(
    pl.kernel,
    out_type=jax.ShapeDtypeStruct((VPAD * DPAD // 128, 128), jnp.float32),
    mesh=_mesh,
    scratch_types=[
        pltpu.VMEM((DPAD, 128), jnp.float32),
        pltpu.VMEM((DPAD, 128), jnp.float32),
        pltpu.VMEM((HALF_ROWS, 128), jnp.float32),
        pltpu.VMEM((HALF_ROWS, 128), jnp.float32),
        pltpu.SemaphoreType.DMA,
        pltpu.SemaphoreType.DMA,
        pltpu.SemaphoreType.DMA,
        pltpu.SemaphoreType.DMA,
    ],
    compiler_params=pltpu.CompilerParams(
        use_tc_tiling_on_sc=True, needs_layout_passes=False
    ),
)
def _sc_transpose(tt_hbm, out_hbm, sbuf0, sbuf1, rbuf0, rbuf1,
                  lsem0, lsem1, wsem0, wsem1):
    wid = lax.axis_index("s") * NUM_CORES + lax.axis_index("c")
    iota = lax.iota(jnp.int32, 16)

    def strip_copies(rt, sbuf, lsem, width):
        descs = []
        for dt in range(NDT):
            rows = 8 if dt < NDT - 1 else 4
            descs.append(pltpu.make_async_copy(
                tt_hbm.at[pl.ds(8 * dt, rows), pl.ds(128 * rt, width)],
                sbuf.at[pl.ds(8 * dt, rows), pl.ds(0, width)],
                lsem,
            ))
        return descs

    def load_strip(rt, sbuf, lsem, width=128):
        for d in strip_copies(rt, sbuf, lsem, width):
            d.start()

    def wait_strip(rt, sbuf, lsem, width=128):
        for d in strip_copies(rt, sbuf, lsem, width):
            d.wait()

    rot = [(iota + s) & 15 for s in range(16)]

    def transpose_half(sbuf, rbuf, col0, ncols=64):
        @plsc.parallel_loop(0, (ncols // 16) * (DPAD // 16), unroll=2)
        def _(i):
            l0 = (i // (DPAD // 16)) * 16
            d0 = (i % (DPAD // 16)) * 16
            lcol = col0 + l0 + iota
            f0 = (l0 + iota) * DPAD + d0
            for s in range(16):
                dv = d0 + rot[s]
                vec = plsc.load_gather(sbuf, [dv, lcol])
                f = f0 + rot[s]
                plsc.store_scatter(rbuf, [f >> 7, f & 127], vec)

    def wb_desc(rt, half, rbuf, wsem, nrows=HALF_ROWS):
        return pltpu.make_async_copy(
            rbuf.at[pl.ds(0, nrows)],
            out_hbm.at[pl.ds(DPAD * rt + HALF_ROWS * half, nrows)],
            wsem,
        )

    def step(t, sbuf, lsem):
        rt = wid + NW * t

        @pl.when(rt < NVT - 1)
        def _():
            wait_strip(rt, sbuf, lsem)

            @pl.when(t >= 1)
            def _():
                wb_desc(0, 0, rbuf0, wsem0).wait()

            transpose_half(sbuf, rbuf0, 0)
            wb_desc(rt, 0, rbuf0, wsem0).start()

            @pl.when(t >= 1)
            def _():
                wb_desc(0, 0, rbuf1, wsem1).wait()

            transpose_half(sbuf, rbuf1, 64)
            wb_desc(rt, 1, rbuf1, wsem1).start()

            @pl.when(rt + 2 * NW < NVT - 1)
            def _():
                load_strip(rt + 2 * NW, sbuf, lsem)

            @pl.when(rt + 2 * NW == NVT - 1)
            def _():
                load_strip(rt + 2 * NW, sbuf, lsem, width=LASTW)

        @pl.when(rt == NVT - 1)
        def _():
            wait_strip(rt, sbuf, lsem, width=LASTW)

            @pl.when(t >= 1)
            def _():
                wb_desc(0, 0, rbuf0, wsem0).wait()

            transpose_half(sbuf, rbuf0, 0, ncols=LASTW)
            wb_desc(rt, 0, rbuf0, wsem0, nrows=LAST_WB_ROWS).start()

    load_strip(wid, sbuf0, lsem0)
    load_strip(wid + NW, sbuf1, lsem1)

    def body(u, _):
        t = 2 * u
        step(t, sbuf0, lsem0)
        step(t + 1, sbuf1, lsem1)
        return 0

    lax.fori_loop(0, (STRIPS_PER_W + 1) // 2, body, 0)

    last0 = wid + NW * (STRIPS_PER_W - 1)

    @pl.when(last0 == NVT - 1)
    def _():
        wb_desc(0, 0, rbuf0, wsem0, nrows=LAST_WB_ROWS).wait()

    @pl.when(last0 != NVT - 1)
    def _():
        wb_desc(0, 0, rbuf0, wsem0).wait()

    wb_desc(0, 0, rbuf1, wsem1).wait()


@functools.partial(
    pl.kernel,
    out_type=jax.ShapeDtypeStruct((HIST, NDT, NBT, 8, 128), jnp.float32),
    mesh=_mesh,
    scratch_types=[
        pltpu.VMEM((64, 128), jnp.int32),
        pltpu.VMEM((16,), jnp.int32),
        pltpu.VMEM((64, DPAD), jnp.float32),
        pltpu.VMEM((64, DPAD), jnp.float32),
        pltpu.VMEM((NDT, 8, 128), jnp.float32),
        pltpu.VMEM((NDT, 8, 128), jnp.float32),
        pltpu.SemaphoreType.DMA,
        pltpu.SemaphoreType.DMA,
        pltpu.SemaphoreType.DMA,
        pltpu.SemaphoreType.DMA,
        pltpu.SemaphoreType.DMA,
    ],
    compiler_params=pltpu.CompilerParams(
        use_tc_tiling_on_sc=False, needs_layout_passes=False
    ),
)
def _sc_gather(xv_hbm, tab_hbm, out_hbm,
               idx_v, hsel, buf0, buf1, cbuf0, cbuf1,
               isem, gsem0, gsem1, ssem0, ssem1):
    wid = lax.axis_index("s") * NUM_CORES + lax.axis_index("c")
    iota = lax.iota(jnp.int32, 16)

    for v in range(4):
        h = 16 * v + iota
        lanes = (h >> 3) * (NBT * 8) + 8 * wid + (h & 7)
        lanes = jnp.where(h < HIST, lanes, 0)
        plsc.store_scatter(hsel, [iota], lanes)
        pltpu.async_copy(
            xv_hbm.at[hsel], idx_v.at[pl.ds(16 * v, 16)], isem
        )
        pltpu.make_async_copy(
            xv_hbm.at[hsel], idx_v.at[pl.ds(16 * v, 16)], isem
        ).wait()

    i_dt = iota >> 3
    i_row = iota & 7

    def gdesc(h, q, buf, gsem):
        return pltpu.make_async_copy(
            tab_hbm.at[idx_v.at[h, pl.ds(64 * q, 64)]], buf, gsem
        )

    rot = [(iota + s) & 15 for s in range(16)]
    rot_hi = [r >> 3 for r in rot]
    rot_lo = [r & 7 for r in rot]

    def transpose_half(buf, cbuf, col0):
        @plsc.parallel_loop(0, 4 * (DPAD // 16), unroll=2)
        def _(i):
            l0 = (i // (DPAD // 16)) * 16
            d0 = (i % (DPAD // 16)) * 16
            lrow = l0 + iota
            lv = col0 + l0 + iota
            d0h = d0 >> 3
            for s in range(16):
                vec = plsc.load_gather(buf, [lrow, d0 + rot[s]])
                plsc.store_scatter(cbuf, [d0h + rot_hi[s], rot_lo[s], lv], vec)

    def wb_desc(h, cbuf, ssem):
        return pltpu.make_async_copy(cbuf, out_hbm.at[h, :, wid], ssem)

    def step(h, cbuf, ssem):
        gdesc(h, 0, buf0, gsem0).wait()

        @pl.when(h >= 2)
        def _():
            wb_desc(h - 2, cbuf, ssem).wait()

        transpose_half(buf0, cbuf, 0)

        @pl.when(h + 1 < HIST)
        def _():
            gdesc(h + 1, 0, buf0, gsem0).start()

        gdesc(h, 1, buf1, gsem1).wait()
        transpose_half(buf1, cbuf, 64)

        @pl.when(h + 1 < HIST)
        def _():
            gdesc(h + 1, 1, buf1, gsem1).start()

        wb_desc(h, cbuf, ssem).start()

    gdesc(0, 0, buf0, gsem0).start()
    gdesc(0, 1, buf1, gsem1).start()

    def body(u, _):
        step(2 * u, cbuf0, ssem0)
        step(2 * u + 1, cbuf1, ssem1)
        return 0

    lax.fori_loop(0, HIST // 2, body, 0)

    wb_desc(HIST - 2, cbuf0, ssem0).wait()
    wb_desc(HIST - 1, cbuf1, ssem1).wait()


def kernel(x, table):
    tab_rm = _sc_transpose(table.T)
    tab_rows = tab_rm.reshape(VPAD, DPAD)

    xt = jnp.pad(x.T.astype(jnp.int32), ((0, 8 * NHT - HIST), (0, 0)))
    xv = (xt.reshape(NHT, 8, NBT, 128)
            .transpose(0, 2, 1, 3)
            .reshape(NHT * NBT * 8, 128))

    o5 = _sc_gather(xv, tab_rows)
    out = o5.transpose(2, 4, 0, 1, 3).reshape(BATCH, HIST, DPAD)
    return out[..., :EMBED_DIM]


# speedup vs baseline: 1.0582x; 1.0582x over previous
import functools

import jax
import jax.numpy as jnp
from jax import lax
from jax.experimental import pallas as pl
from jax.experimental.pallas import tpu as pltpu
from jax.experimental.pallas import tpu_sc as plsc

VOCAB = 100000
EMBED_DIM = 300
BATCH = 4096
HIST = 50

NUM_CORES = 2
NUM_SUBCORES = 16
NW = NUM_CORES * NUM_SUBCORES

VPAD = 100096
DPAD = 304
NDT = DPAD // 8
NVT = VPAD // 128
LASTW = VOCAB - 128 * (NVT - 1)
HALF_ROWS = 64 * DPAD // 128
LAST_WB_ROWS = 80
NBT = BATCH // 128
NHT = 7

STRIPS_PER_W = 25

_mesh = plsc.VectorSubcoreMesh(core_axis_name="c", subcore_axis_name="s")


@functools.partial(
    pl.kernel,
    out_type=jax.ShapeDtypeStruct((VPAD * DPAD // 128, 128), jnp.float32),
    mesh=_mesh,
    scratch_types=[
        pltpu.VMEM((DPAD, 128), jnp.float32),
        pltpu.VMEM((DPAD, 128), jnp.float32),
        pltpu.VMEM((HALF_ROWS, 128), jnp.float32),
        pltpu.VMEM((HALF_ROWS, 128), jnp.float32),
        pltpu.SemaphoreType.DMA,
        pltpu.SemaphoreType.DMA,
        pltpu.SemaphoreType.DMA,
        pltpu.SemaphoreType.DMA,
    ],
    compiler_params=pltpu.CompilerParams(
        use_tc_tiling_on_sc=True, needs_layout_passes=False
    ),
)
def _sc_transpose(tt_hbm, out_hbm, sbuf0, sbuf1, rbuf0, rbuf1,
                  lsem0, lsem1, wsem0, wsem1):
    wid = lax.axis_index("s") * NUM_CORES + lax.axis_index("c")
    iota = lax.iota(jnp.int32, 16)

    def strip_copies(rt, sbuf, lsem, width):
        descs = []
        for dt in range(NDT):
            rows = 8 if dt < NDT - 1 else 4
            descs.append(pltpu.make_async_copy(
                tt_hbm.at[pl.ds(8 * dt, rows), pl.ds(128 * rt, width)],
                sbuf.at[pl.ds(8 * dt, rows), pl.ds(0, width)],
                lsem,
            ))
        return descs

    def load_strip(rt, sbuf, lsem, width=128):
        for d in strip_copies(rt, sbuf, lsem, width):
            d.start()

    def wait_strip(rt, sbuf, lsem, width=128):
        for d in strip_copies(rt, sbuf, lsem, width):
            d.wait()

    rot = [(iota + s) & 15 for s in range(16)]

    def transpose_half(sbuf, rbuf, col0, ncols=64):
        @plsc.parallel_loop(0, (ncols // 16) * (DPAD // 16))
        def _(i):
            l0 = (i // (DPAD // 16)) * 16
            d0 = (i % (DPAD // 16)) * 16
            lcol = col0 + l0 + iota
            f0 = (l0 + iota) * DPAD + d0
            for s in range(16):
                dv = d0 + rot[s]
                vec = plsc.load_gather(sbuf, [dv, lcol])
                f = f0 + rot[s]
                plsc.store_scatter(rbuf, [f >> 7, f & 127], vec)

    def wb_desc(rt, half, rbuf, wsem, nrows=HALF_ROWS):
        return pltpu.make_async_copy(
            rbuf.at[pl.ds(0, nrows)],
            out_hbm.at[pl.ds(DPAD * rt + HALF_ROWS * half, nrows)],
            wsem,
        )

    def step(t, sbuf, lsem):
        rt = wid + NW * t

        @pl.when(rt < NVT - 1)
        def _():
            wait_strip(rt, sbuf, lsem)

            @pl.when(t >= 1)
            def _():
                wb_desc(0, 0, rbuf0, wsem0).wait()

            transpose_half(sbuf, rbuf0, 0)
            wb_desc(rt, 0, rbuf0, wsem0).start()

            @pl.when(t >= 1)
            def _():
                wb_desc(0, 0, rbuf1, wsem1).wait()

            transpose_half(sbuf, rbuf1, 64)
            wb_desc(rt, 1, rbuf1, wsem1).start()

            @pl.when(rt + 2 * NW < NVT - 1)
            def _():
                load_strip(rt + 2 * NW, sbuf, lsem)

            @pl.when(rt + 2 * NW == NVT - 1)
            def _():
                load_strip(rt + 2 * NW, sbuf, lsem, width=LASTW)

        @pl.when(rt == NVT - 1)
        def _():
            wait_strip(rt, sbuf, lsem, width=LASTW)

            @pl.when(t >= 1)
            def _():
                wb_desc(0, 0, rbuf0, wsem0).wait()

            transpose_half(sbuf, rbuf0, 0, ncols=LASTW)
            wb_desc(rt, 0, rbuf0, wsem0, nrows=LAST_WB_ROWS).start()

    load_strip(wid, sbuf0, lsem0)
    load_strip(wid + NW, sbuf1, lsem1)

    def body(u, _):
        t = 2 * u
        step(t, sbuf0, lsem0)
        step(t + 1, sbuf1, lsem1)
        return 0

    lax.fori_loop(0, (STRIPS_PER_W + 1) // 2, body, 0)

    last0 = wid + NW * (STRIPS_PER_W - 1)

    @pl.when(last0 == NVT - 1)
    def _():
        wb_desc(0, 0, rbuf0, wsem0, nrows=LAST_WB_ROWS).wait()

    @pl.when(last0 != NVT - 1)
    def _():
        wb_desc(0, 0, rbuf0, wsem0).wait()

    wb_desc(0, 0, rbuf1, wsem1).wait()


@functools.partial(
    pl.kernel,
    out_type=jax.ShapeDtypeStruct((HIST, NDT, NBT, 8, 128), jnp.float32),
    mesh=_mesh,
    scratch_types=[
        pltpu.VMEM((64, 128), jnp.int32),
        pltpu.VMEM((16,), jnp.int32),
        pltpu.VMEM((64, DPAD), jnp.float32),
        pltpu.VMEM((64, DPAD), jnp.float32),
        pltpu.VMEM((NDT, 8, 128), jnp.float32),
        pltpu.VMEM((NDT, 8, 128), jnp.float32),
        pltpu.SemaphoreType.DMA,
        pltpu.SemaphoreType.DMA,
        pltpu.SemaphoreType.DMA,
        pltpu.SemaphoreType.DMA,
        pltpu.SemaphoreType.DMA,
    ],
    compiler_params=pltpu.CompilerParams(
        use_tc_tiling_on_sc=False, needs_layout_passes=False
    ),
)
def _sc_gather(xv_hbm, tab_hbm, out_hbm,
               idx_v, hsel, buf0, buf1, cbuf0, cbuf1,
               isem, gsem0, gsem1, ssem0, ssem1):
    wid = lax.axis_index("s") * NUM_CORES + lax.axis_index("c")
    iota = lax.iota(jnp.int32, 16)

    for v in range(4):
        h = 16 * v + iota
        lanes = (h >> 3) * (NBT * 8) + 8 * wid + (h & 7)
        lanes = jnp.where(h < HIST, lanes, 0)
        plsc.store_scatter(hsel, [iota], lanes)
        pltpu.async_copy(
            xv_hbm.at[hsel], idx_v.at[pl.ds(16 * v, 16)], isem
        )
        pltpu.make_async_copy(
            xv_hbm.at[hsel], idx_v.at[pl.ds(16 * v, 16)], isem
        ).wait()

    i_dt = iota >> 3
    i_row = iota & 7

    def gdesc(h, q, buf, gsem):
        return pltpu.make_async_copy(
            tab_hbm.at[idx_v.at[h, pl.ds(64 * q, 64)]], buf, gsem
        )

    rot = [(iota + s) & 15 for s in range(16)]
    rot_hi = [r >> 3 for r in rot]
    rot_lo = [r & 7 for r in rot]

    def transpose_half(buf, cbuf, col0):
        @plsc.parallel_loop(0, 4 * (DPAD // 16), unroll=2)
        def _(i):
            l0 = (i // (DPAD // 16)) * 16
            d0 = (i % (DPAD // 16)) * 16
            lrow = l0 + iota
            lv = col0 + l0 + iota
            d0h = d0 >> 3
            for s in range(16):
                vec = plsc.load_gather(buf, [lrow, d0 + rot[s]])
                plsc.store_scatter(cbuf, [d0h + rot_hi[s], rot_lo[s], lv], vec)

    def wb_desc(h, cbuf, ssem):
        return pltpu.make_async_copy(cbuf, out_hbm.at[h, :, wid], ssem)

    def step(h, cbuf, ssem):
        gdesc(h, 0, buf0, gsem0).wait()

        @pl.when(h >= 2)
        def _():
            wb_desc(h - 2, cbuf, ssem).wait()

        transpose_half(buf0, cbuf, 0)

        @pl.when(h + 1 < HIST)
        def _():
            gdesc(h + 1, 0, buf0, gsem0).start()

        gdesc(h, 1, buf1, gsem1).wait()
        transpose_half(buf1, cbuf, 64)

        @pl.when(h + 1 < HIST)
        def _():
            gdesc(h + 1, 1, buf1, gsem1).start()

        wb_desc(h, cbuf, ssem).start()

    gdesc(0, 0, buf0, gsem0).start()
    gdesc(0, 1, buf1, gsem1).start()

    def body(u, _):
        step(2 * u, cbuf0, ssem0)
        step(2 * u + 1, cbuf1, ssem1)
        return 0

    lax.fori_loop(0, HIST // 2, body, 0)

    wb_desc(HIST - 2, cbuf0, ssem0).wait()
    wb_desc(HIST - 1, cbuf1, ssem1).wait()


def kernel(x, table):
    tab_rm = _sc_transpose(table.T)
    tab_rows = tab_rm.reshape(VPAD, DPAD)

    xt = jnp.pad(x.T.astype(jnp.int32), ((0, 8 * NHT - HIST), (0, 0)))
    xv = (xt.reshape(NHT, 8, NBT, 128)
            .transpose(0, 2, 1, 3)
            .reshape(NHT * NBT * 8, 128))

    o5 = _sc_gather(xv, tab_rows)
    out = o5.transpose(2, 4, 0, 1, 3).reshape(BATCH, HIST, DPAD)
    return out[..., :EMBED_DIM]
